# baseline (speedup 1.0000x reference)
import functools

import jax
import jax.numpy as jnp
from jax import lax
from jax.experimental import pallas as pl
from jax.experimental.pallas import tpu as pltpu

_MIB = 1 << 20
_S = 4


def _se_pipeline(x_hbm, w1_ref, w2_ref, o_hbm, *bufs, n_steps, inv_hw):
    x_bufs = bufs[0:_S]
    o_bufs = bufs[_S:2 * _S]
    in_sems = bufs[2 * _S:3 * _S]
    out_sems = bufs[3 * _S:4 * _S]

    def start_in(step, slot):
        pltpu.make_async_copy(x_hbm.at[step], x_bufs[slot],
                              in_sems[slot]).start()

    def wait_in(slot):
        pltpu.make_async_copy(x_hbm.at[0], x_bufs[slot],
                              in_sems[slot]).wait()

    def start_out(step, slot):
        pltpu.make_async_copy(o_bufs[slot], o_hbm.at[step],
                              out_sems[slot]).start()

    def wait_out(slot):
        pltpu.make_async_copy(o_bufs[slot], o_hbm.at[0],
                              out_sems[slot]).wait()

    for p in range(_S):
        start_in(p, p)

    def outer_body(outer, _):
        for slot in range(_S):
            step = outer * _S + slot
            wait_in(slot)

            @pl.when(step >= _S)
            def _():
                wait_out(slot)

            x = x_bufs[slot][...]
            pooled = jnp.sum(x, axis=1, keepdims=True,
                             dtype=jnp.float32) * inv_hw
            h = lax.dot_general(w1_ref[...], pooled,
                                (((1,), (0,)), ((), ())),
                                preferred_element_type=jnp.float32)
            h = jnp.maximum(h, 0.0)
            s = lax.dot_general(w2_ref[...], h,
                                (((1,), (0,)), ((), ())),
                                preferred_element_type=jnp.float32)
            s = jax.nn.sigmoid(s).astype(x.dtype)
            o_bufs[slot][...] = x * s
            start_out(step, slot)

            @pl.when(step + _S < n_steps)
            def _():
                start_in(step + _S, slot)
        return ()

    lax.fori_loop(0, n_steps // _S, outer_body, ())
    for p in range(_S):
        wait_out(p)


def kernel(x, w1, w2):
    B, C, H, W = x.shape
    HW = H * W
    inv_hw = 1.0 / float(HW)
    x3 = x.reshape(B, C, HW)

    buf_bytes = 2 * _S * C * HW * x.dtype.itemsize
    vmem_limit = int(min(63 * _MIB, buf_bytes + 8 * _MIB))
    scratch = ([pltpu.VMEM((C, HW), x.dtype) for _ in range(2 * _S)]
               + [pltpu.SemaphoreType.DMA for _ in range(2 * _S)])
    out3 = pl.pallas_call(
        functools.partial(_se_pipeline, n_steps=B, inv_hw=inv_hw),
        out_shape=jax.ShapeDtypeStruct((B, C, HW), x.dtype),
        in_specs=[
            pl.BlockSpec(memory_space=pl.ANY),
            pl.BlockSpec(memory_space=pltpu.VMEM),
            pl.BlockSpec(memory_space=pltpu.VMEM),
        ],
        out_specs=pl.BlockSpec(memory_space=pl.ANY),
        scratch_shapes=scratch,
        compiler_params=pltpu.CompilerParams(
            vmem_limit_bytes=vmem_limit,
        ),
    )(x3, w1, w2)
    return out3.reshape(B, C, H, W)
